# baseline (speedup 1.0000x reference)
import functools

import jax
import jax.numpy as jnp
from jax import lax
from jax.experimental import pallas as pl
from jax.experimental.pallas import tpu as pltpu
from jax.experimental.pallas import tpu_sc as plsc

NC = 2
NS = 16
L = 16
NW = NC * NS

B = 4096
CH = 96
NB = 48


def _sc_segment_embed(w0t, vals_p, cols_p, segs_p, bounds):
    V, D1 = w0t.shape
    rows_per = B // NW
    DC = D1 // L

    mesh = plsc.VectorSubcoreMesh(core_axis_name="c", subcore_axis_name="s")

    @functools.partial(
        pl.kernel,
        out_type=jax.ShapeDtypeStruct((B, D1), jnp.float32),
        mesh=mesh,
        scratch_types=[
            pltpu.VMEM((NB,), jnp.int32),
            [pltpu.VMEM((CH,), jnp.int32) for _ in range(3)],
            [pltpu.VMEM((CH,), jnp.int32) for _ in range(3)],
            [pltpu.VMEM((CH,), jnp.float32) for _ in range(3)],
            [pltpu.VMEM((CH, D1), jnp.float32) for _ in range(3)],
            pltpu.VMEM((rows_per, D1), jnp.float32),
            [pltpu.SemaphoreType.DMA for _ in range(3)],
            [pltpu.SemaphoreType.DMA for _ in range(3)],
        ],
    )
    def sc_kernel(w0t_hbm, vals_hbm, cols_hbm, segs_hbm, bounds_hbm, out_hbm,
                  boundsv, cols3, segs3, vals3, rows3, acc, gsems, isems):
        cid = lax.axis_index("c")
        sid = lax.axis_index("s")
        wid = cid * NS + sid
        s0 = wid * rows_per

        pltpu.sync_copy(bounds_hbm, boundsv)
        bvec = boundsv[pl.ds(wid, L)]
        start = bvec[0]
        end = bvec[1]
        start_al = (start // 8) * 8
        n_ch = (end - start_al + CH - 1) // CH

        lane = lax.iota(jnp.int32, L)
        zero = jnp.zeros((L,), jnp.float32)

        def zrow(j, c0):
            for c in range(DC):
                acc[j, c * L:(c + 1) * L] = zero
            return c0
        lax.fori_loop(0, rows_per, zrow, 0)

        def load_idx(t, colv, segv, valv, isem):
            off = start_al + t * CH
            pltpu.async_copy(cols_hbm.at[pl.ds(off, CH)], colv, isem)
            pltpu.async_copy(segs_hbm.at[pl.ds(off, CH)], segv, isem)
            pltpu.async_copy(vals_hbm.at[pl.ds(off, CH)], valv, isem)

        def wait_idx(colv, segv, valv, isem):
            pltpu.make_async_copy(cols_hbm.at[pl.ds(0, CH)], colv, isem).wait()
            pltpu.make_async_copy(segs_hbm.at[pl.ds(0, CH)], segv, isem).wait()
            pltpu.make_async_copy(vals_hbm.at[pl.ds(0, CH)], valv, isem).wait()

        def start_gather(colv, rowsv, gsem):
            pltpu.async_copy(w0t_hbm.at[colv], rowsv, gsem)

        def wait_gather(colv, rowsv, gsem):
            pltpu.make_async_copy(w0t_hbm.at[colv], rowsv, gsem).wait()

        def accum_chunk(t, segv, valv, rowsv):
            off = start_al + t * CH

            def group(g, c0):
                j0 = g * L
                segs16 = segv[pl.ds(j0, L)]
                vals16 = valv[pl.ds(j0, L)]
                jglob = off + j0 + lane
                m = (jglob >= start) & (jglob < end)
                vals16 = jnp.where(m, vals16, 0.0)
                r16 = jnp.clip(segs16 - s0, 0, rows_per - 1)
                ra = r16[0]
                rb = r16[L - 1]
                srun = ra == rb
                ra_v = jnp.full((L,), ra, jnp.int32)
                rb_v = jnp.full((L,), rb, jnp.int32)
                m_a = r16 == ra_v
                good = (r16 - ra_v) * (rb_v - r16)
                anybad = good[1]
                for k in range(2, L - 1):
                    anybad = anybad | good[k]
                tworun = jnp.logical_and(jnp.logical_not(srun),
                                         anybad == 0)

                @pl.when(srun)
                def _():
                    gsum = list(zero for _ in range(DC))
                    for k in range(L):
                        vv = jnp.full((L,), vals16[k], jnp.float32)
                        j = j0 + k
                        for c in range(DC):
                            gsum[c] = gsum[c] + rowsv[j, pl.ds(c * L, L)] * vv
                    for c in range(DC):
                        sl = pl.ds(c * L, L)
                        acc[ra, sl] = acc[ra, sl] + gsum[c]

                @pl.when(tworun)
                def _():
                    va = jnp.where(m_a, vals16, 0.0)
                    tsum = list(zero for _ in range(DC))
                    asum = list(zero for _ in range(DC))
                    for k in range(L):
                        vv = jnp.full((L,), vals16[k], jnp.float32)
                        vva = jnp.full((L,), va[k], jnp.float32)
                        j = j0 + k
                        for c in range(DC):
                            row = rowsv[j, pl.ds(c * L, L)]
                            tsum[c] = tsum[c] + row * vv
                            asum[c] = asum[c] + row * vva
                    for c in range(DC):
                        sl = pl.ds(c * L, L)
                        acc[ra, sl] = acc[ra, sl] + asum[c]
                        acc[rb, sl] = acc[rb, sl] + (tsum[c] - asum[c])

                @pl.when(jnp.logical_not(jnp.logical_or(srun, tworun)))
                def _():
                    for k in range(L):
                        r = r16[k]
                        vv = jnp.full((L,), vals16[k], jnp.float32)
                        j = j0 + k
                        for c in range(DC):
                            sl = pl.ds(c * L, L)
                            acc[r, sl] = acc[r, sl] + rowsv[j, sl] * vv
                return c0
            lax.fori_loop(0, CH // L, group, 0)

        load_idx(0, cols3[0], segs3[0], vals3[0], isems[0])
        load_idx(1, cols3[1], segs3[1], vals3[1], isems[1])
        load_idx(2, cols3[2], segs3[2], vals3[2], isems[2])
        wait_idx(cols3[0], segs3[0], vals3[0], isems[0])
        start_gather(cols3[0], rows3[0], gsems[0])

        def step(t, cur, nxt):
            wait_idx(cols3[nxt], segs3[nxt], vals3[nxt], isems[nxt])
            start_gather(cols3[nxt], rows3[nxt], gsems[nxt])
            wait_gather(cols3[cur], rows3[cur], gsems[cur])
            accum_chunk(t, segs3[cur], vals3[cur], rows3[cur])
            load_idx(t + 3, cols3[cur], segs3[cur], vals3[cur], isems[cur])

        def triple(u, c0):
            t0 = 3 * u
            step(t0, 0, 1)
            step(t0 + 1, 1, 2)
            step(t0 + 2, 2, 0)
            return c0
        n_tri = (n_ch + 2) // 3
        lax.fori_loop(0, n_tri, triple, 0)

        wait_gather(cols3[0], rows3[0], gsems[0])
        wait_idx(cols3[1], segs3[1], vals3[1], isems[1])
        wait_idx(cols3[2], segs3[2], vals3[2], isems[2])

        pltpu.sync_copy(acc, out_hbm.at[pl.ds(s0, rows_per)])

    return sc_kernel(w0t, vals_p, cols_p, segs_p, bounds)


def _tc_head(hidden, W1):
    Bv, D1 = hidden.shape
    D2 = W1.shape[0]
    bm = 512

    def body(h_ref, w1_ref, o_ref):
        h = jnp.maximum(h_ref[...], 0.0)
        n = jnp.sqrt(jnp.sum(h * h, axis=1, keepdims=True))
        h = h / jnp.maximum(n, 1e-12)
        o = lax.dot_general(h, w1_ref[...], (((1,), (1,)), ((), ())),
                            preferred_element_type=jnp.float32,
                            precision=lax.Precision.HIGHEST)
        n2 = jnp.sqrt(jnp.sum(o * o, axis=1, keepdims=True))
        o_ref[...] = o / jnp.maximum(n2, 1e-12)

    return pl.pallas_call(
        body,
        grid=(Bv // bm,),
        in_specs=[
            pl.BlockSpec((bm, D1), lambda i: (i, 0)),
            pl.BlockSpec((D2, D1), lambda i: (0, 0)),
        ],
        out_specs=pl.BlockSpec((bm, D2), lambda i: (i, 0)),
        out_shape=jax.ShapeDtypeStruct((Bv, D2), jnp.float32),
    )(hidden, W1)


def kernel(values, col_indices, segment_ids, W0, W1):
    w0t = W0.T
    segment_ids = segment_ids.astype(jnp.int32)
    col_indices = col_indices.astype(jnp.int32)
    nnz = values.shape[0]
    rows_per = B // NW
    targets = jnp.arange(NW + 1, dtype=jnp.int32) * rows_per
    bounds = jnp.searchsorted(segment_ids, targets).astype(jnp.int32)
    bounds = jnp.pad(bounds, (0, NB - (NW + 1)), constant_values=nnz)
    pad = 8 * CH
    vals_p = jnp.pad(values, (0, pad))
    cols_p = jnp.pad(col_indices, (0, pad))
    segs_p = jnp.pad(segment_ids, (0, pad), constant_values=B)
    hidden = _sc_segment_embed(w0t, vals_p, cols_p, segs_p, bounds)
    return _tc_head(hidden, W1)


# speedup vs baseline: 1.1822x; 1.1822x over previous
import functools

import jax
import jax.numpy as jnp
from jax import lax
from jax.experimental import pallas as pl
from jax.experimental.pallas import tpu as pltpu
from jax.experimental.pallas import tpu_sc as plsc

NC = 2
NS = 16
L = 16
NW = NC * NS

B = 4096
CH = 128
NB = 48


def _sc_segment_embed(w0t, vals_p, cols_p, segs_p, bounds):
    V, D1 = w0t.shape
    nnz_al = vals_p.shape[0]
    assert nnz_al % CH == 0
    rows_per = B // NW
    DC = D1 // L

    mesh = plsc.VectorSubcoreMesh(core_axis_name="c", subcore_axis_name="s")

    @functools.partial(
        pl.kernel,
        out_type=jax.ShapeDtypeStruct((B, D1), jnp.float32),
        mesh=mesh,
        scratch_types=[
            pltpu.VMEM((NB,), jnp.int32),
            pltpu.VMEM((CH,), jnp.int32),
            pltpu.VMEM((CH,), jnp.int32),
            pltpu.VMEM((CH,), jnp.int32),
            pltpu.VMEM((CH,), jnp.int32),
            pltpu.VMEM((CH,), jnp.float32),
            pltpu.VMEM((CH,), jnp.float32),
            pltpu.VMEM((CH, D1), jnp.float32),
            pltpu.VMEM((CH, D1), jnp.float32),
            pltpu.VMEM((rows_per, D1), jnp.float32),
            pltpu.SemaphoreType.DMA,
            pltpu.SemaphoreType.DMA,
            pltpu.SemaphoreType.DMA,
            pltpu.SemaphoreType.DMA,
        ],
    )
    def sc_kernel(w0t_hbm, vals_hbm, cols_hbm, segs_hbm, bounds_hbm, out_hbm,
                  boundsv, colA, colB, segA, segB, valA, valB,
                  rowsA, rowsB, acc, gsemA, gsemB, isemA, isemB):
        cid = lax.axis_index("c")
        sid = lax.axis_index("s")
        wid = cid * NS + sid
        s0 = wid * rows_per

        pltpu.sync_copy(bounds_hbm, boundsv)
        bvec = boundsv[pl.ds(wid, L)]
        start = bvec[0]
        end = bvec[1]
        start_al = (start // 8) * 8
        n_ch = (end - start_al + CH - 1) // CH
        n_pairs = (n_ch + 1) // 2

        lane = lax.iota(jnp.int32, L)
        zero = jnp.zeros((L,), jnp.float32)

        def zrow(j, c0):
            for c in range(DC):
                acc[j, c * L:(c + 1) * L] = zero
            return c0
        lax.fori_loop(0, rows_per, zrow, 0)

        def chunk_off(t):
            return jnp.minimum(start_al + t * CH, nnz_al - CH)

        def load_idx(t, colv, segv, valv, isem):
            off = chunk_off(t)
            pltpu.async_copy(cols_hbm.at[pl.ds(off, CH)], colv, isem)
            pltpu.async_copy(segs_hbm.at[pl.ds(off, CH)], segv, isem)
            pltpu.async_copy(vals_hbm.at[pl.ds(off, CH)], valv, isem)

        def wait_idx(colv, segv, valv, isem):
            pltpu.make_async_copy(cols_hbm.at[pl.ds(0, CH)], colv, isem).wait()
            pltpu.make_async_copy(segs_hbm.at[pl.ds(0, CH)], segv, isem).wait()
            pltpu.make_async_copy(vals_hbm.at[pl.ds(0, CH)], valv, isem).wait()

        def start_gather(colv, rowsv, gsem):
            pltpu.async_copy(w0t_hbm.at[colv], rowsv, gsem)

        def wait_gather(colv, rowsv, gsem):
            pltpu.make_async_copy(w0t_hbm.at[colv], rowsv, gsem).wait()

        def accum_chunk(t, segv, valv, rowsv, carry):
            off = chunk_off(t)
            lo_t = start_al + t * CH

            def group(g, carry2):
                cur_r, regs = carry2
                j0 = g * L
                segs16 = segv[pl.ds(j0, L)]
                vals16 = valv[pl.ds(j0, L)]
                jglob = off + j0 + lane
                lo = jnp.maximum(start, lo_t)
                m = (jglob >= lo) & (jglob < end)
                vals16 = jnp.where(m, vals16, 0.0)
                r16 = jnp.clip(segs16 - s0, 0, rows_per - 1)
                for k in range(L):
                    r = r16[k]
                    flush = r != cur_r

                    @pl.when(flush)
                    def _():
                        for c in range(DC):
                            sl = pl.ds(c * L, L)
                            acc[cur_r, sl] = acc[cur_r, sl] + regs[c]

                    vv = jnp.full((L,), vals16[k], jnp.float32)
                    j = j0 + k
                    regs = tuple(
                        jnp.where(flush, 0.0, regs[c])
                        + rowsv[j, pl.ds(c * L, L)] * vv
                        for c in range(DC))
                    cur_r = r
                return cur_r, regs
            return lax.fori_loop(0, CH // L, group, carry)

        load_idx(0, colA, segA, valA, isemA)
        wait_idx(colA, segA, valA, isemA)
        start_gather(colA, rowsA, gsemA)
        load_idx(1, colB, segB, valB, isemB)
        wait_idx(colB, segB, valB, isemB)

        carry0 = (jnp.int32(0), tuple(zero for _ in range(DC)))

        def pair(u, carry):
            tA = 2 * u
            tB = tA + 1
            start_gather(colB, rowsB, gsemB)
            wait_gather(colA, rowsA, gsemA)
            carry = accum_chunk(tA, segA, valA, rowsA, carry)
            load_idx(tA + 2, colA, segA, valA, isemA)
            wait_idx(colA, segA, valA, isemA)
            start_gather(colA, rowsA, gsemA)
            wait_gather(colB, rowsB, gsemB)
            carry = accum_chunk(tB, segB, valB, rowsB, carry)
            load_idx(tB + 2, colB, segB, valB, isemB)
            wait_idx(colB, segB, valB, isemB)
            return carry
        cur_r, regs = lax.fori_loop(0, n_pairs, pair, carry0)

        for c in range(DC):
            sl = pl.ds(c * L, L)
            acc[cur_r, sl] = acc[cur_r, sl] + regs[c]

        wait_gather(colA, rowsA, gsemA)

        pltpu.sync_copy(acc, out_hbm.at[pl.ds(s0, rows_per)])

    return sc_kernel(w0t, vals_p, cols_p, segs_p, bounds)


def _tc_head(hidden, W1):
    Bv, D1 = hidden.shape
    D2 = W1.shape[0]
    bm = 512

    def body(h_ref, w1_ref, o_ref):
        h = jnp.maximum(h_ref[...], 0.0)
        n = jnp.sqrt(jnp.sum(h * h, axis=1, keepdims=True))
        h = h / jnp.maximum(n, 1e-12)
        o = lax.dot_general(h, w1_ref[...], (((1,), (1,)), ((), ())),
                            preferred_element_type=jnp.float32,
                            precision=lax.Precision.HIGHEST)
        n2 = jnp.sqrt(jnp.sum(o * o, axis=1, keepdims=True))
        o_ref[...] = o / jnp.maximum(n2, 1e-12)

    return pl.pallas_call(
        body,
        grid=(Bv // bm,),
        in_specs=[
            pl.BlockSpec((bm, D1), lambda i: (i, 0)),
            pl.BlockSpec((D2, D1), lambda i: (0, 0)),
        ],
        out_specs=pl.BlockSpec((bm, D2), lambda i: (i, 0)),
        out_shape=jax.ShapeDtypeStruct((Bv, D2), jnp.float32),
    )(hidden, W1)


def kernel(values, col_indices, segment_ids, W0, W1):
    w0t = W0.T
    segment_ids = segment_ids.astype(jnp.int32)
    col_indices = col_indices.astype(jnp.int32)
    nnz = values.shape[0]
    rows_per = B // NW
    targets = jnp.arange(NW + 1, dtype=jnp.int32) * rows_per
    bounds = jnp.searchsorted(segment_ids, targets).astype(jnp.int32)
    bounds = jnp.pad(bounds, (0, NB - (NW + 1)), constant_values=nnz)
    hidden = _sc_segment_embed(w0t, values, col_indices, segment_ids, bounds)
    return _tc_head(hidden, W1)
